# Initial kernel scaffold; baseline (speedup 1.0000x reference)
import jax
import jax.numpy as jnp
from jax.experimental import pallas as pl


def kernel(x, edge_index, params):
    raise NotImplementedError("write your pallas kernel here")


# speedup vs baseline: 1.3533x; 1.3533x over previous
import functools

import jax
import jax.numpy as jnp
from jax import lax
from jax.experimental import pallas as pl
from jax.experimental.pallas import tpu as pltpu
from jax.experimental.pallas import tpu_sc as plsc

N = 50000
E = 800000
EPS = 1e-5
CHUNK = 128

EPW = E // 32
GFULL = EPW // CHUNK
GTAIL = EPW - GFULL * CHUNK

EPSUB = E // 16
SFULL = EPSUB // CHUNK
STAIL = EPSUB - SFULL * CHUNK

ZCH = 25
SPR = 16 * ZCH * CHUNK


def _fs(shape):
    return jax.ShapeDtypeStruct(shape, jnp.float32)


_MESH = plsc.VectorSubcoreMesh(core_axis_name="c", subcore_axis_name="s")


def _g_sc(tab_hbm, dst_hbm, src_hbm, ga_hbm, gb_hbm,
          div, siv, av, bv, div40, siv40, av40, bv40, sem_a, sem_b):
    c = lax.axis_index("c")
    s = lax.axis_index("s")
    base = (s * 2 + c) * EPW

    def chunk(k, _):
        o = base + k * CHUNK
        pltpu.sync_copy(dst_hbm.at[pl.ds(o, CHUNK)], div)
        pltpu.sync_copy(src_hbm.at[pl.ds(o, CHUNK)], siv)
        ca = pltpu.async_copy(tab_hbm.at[div], av, sem_a)
        cb = pltpu.async_copy(tab_hbm.at[siv], bv, sem_b)
        ca.wait()
        cb.wait()
        pltpu.sync_copy(av, ga_hbm.at[pl.ds(o, CHUNK)])
        pltpu.sync_copy(bv, gb_hbm.at[pl.ds(o, CHUNK)])
        return 0

    lax.fori_loop(0, GFULL, chunk, 0)

    o = base + GFULL * CHUNK
    pltpu.sync_copy(dst_hbm.at[pl.ds(o, GTAIL)], div40)
    pltpu.sync_copy(src_hbm.at[pl.ds(o, GTAIL)], siv40)
    ca = pltpu.async_copy(tab_hbm.at[div40], av40, sem_a)
    cb = pltpu.async_copy(tab_hbm.at[siv40], bv40, sem_b)
    ca.wait()
    cb.wait()
    pltpu.sync_copy(av40, ga_hbm.at[pl.ds(o, GTAIL)])
    pltpu.sync_copy(bv40, gb_hbm.at[pl.ds(o, GTAIL)])


def _g_call(tab, dst, src):
    return pl.kernel(
        _g_sc,
        mesh=_MESH,
        out_type=[_fs((E, 128)), _fs((E, 128))],
        scratch_types=[
            pltpu.VMEM((CHUNK,), jnp.int32),
            pltpu.VMEM((CHUNK,), jnp.int32),
            pltpu.VMEM((CHUNK, 128), jnp.float32),
            pltpu.VMEM((CHUNK, 128), jnp.float32),
            pltpu.VMEM((GTAIL,), jnp.int32),
            pltpu.VMEM((GTAIL,), jnp.int32),
            pltpu.VMEM((GTAIL, 128), jnp.float32),
            pltpu.VMEM((GTAIL, 128), jnp.float32),
            pltpu.SemaphoreType.DMA,
            pltpu.SemaphoreType.DMA,
        ],
    )(tab, dst, src)


def _rows(c, bs):
    return pl.BlockSpec((bs, c), lambda i: (i, 0))


def _full(shape):
    return pl.BlockSpec(shape, lambda i: tuple(0 for _ in shape))


def _acc_spec(c, bs):
    return pl.BlockSpec((2, bs, c), lambda i: (0, i, 0))


def _stats_fold(ssum, ssq, n, g, be):
    mu = ssum / n
    var = ssq / n - mu * mu
    s = g / jnp.sqrt(var + EPS)
    return s, be - mu * s


def _p1_body(x_ref, wd_ref, ws_ref, b_ref, t_ref):
    xb = x_ref[...]
    t_ref[...] = jnp.concatenate(
        [xb @ wd_ref[...] + b_ref[0], xb @ ws_ref[...]], axis=1)


def _p1(x, wdt, wst, b):
    return pl.pallas_call(
        _p1_body,
        grid=(10,),
        in_specs=[_rows(4, 5000), _full((4, 64)), _full((4, 64)),
                  _full((1, 64))],
        out_specs=_rows(128, 5000),
        out_shape=_fs((N, 128)),
    )(x, wdt, wst, b)


def _p23_body(acc_ref, degp_ref, st_ref, g_ref, be_ref, wd_ref, ws_ref, b_ref,
              x_ref, t_ref):
    p = st_ref[...]
    s2, t2 = _stats_fold(p[0], p[1], float(E), g_ref[0], be_ref[0])
    deg = degp_ref[0, :, 0:1] + degp_ref[1, :, 0:1]
    summed = jnp.concatenate([acc_ref[0], acc_ref[1]], axis=1)
    xb = summed * s2 + deg * t2[None, :]
    x_ref[...] = xb
    t_ref[...] = jnp.concatenate(
        [xb @ wd_ref[...] + b_ref[0], xb @ ws_ref[...]], axis=1)


def _p23(acc, degp, st2, g2, be2, wdt, wst, b):
    return pl.pallas_call(
        _p23_body,
        grid=(10,),
        in_specs=[_acc_spec(32, 5000), _acc_spec(16, 5000),
                  _full((8, 64)), _full((1, 64)), _full((1, 64)),
                  _full((64, 64)), _full((64, 64)), _full((1, 64))],
        out_specs=[_rows(64, 5000), _rows(128, 5000)],
        out_shape=[_fs((N, 64)), _fs((N, 128))],
    )(acc, degp, st2, g2, be2, wdt, wst, b)


def _m1_body(ga_ref, gb_ref, st_ref, acc_ref, *, nsteps):
    i = pl.program_id(0)

    @pl.when(i == 0)
    def _():
        acc_ref[...] = jnp.zeros_like(acc_ref)

    h = jnp.maximum(ga_ref[:, :64] + gb_ref[:, 64:], 0.0)
    acc_ref[...] += jnp.concatenate(
        [jnp.sum(h, axis=0, keepdims=True),
         jnp.sum(h * h, axis=0, keepdims=True),
         jnp.zeros((6, 64), jnp.float32)], axis=0)

    @pl.when(i == nsteps - 1)
    def _():
        st_ref[...] = acc_ref[...]


def _m1(ga, gb):
    nsteps = 100
    return pl.pallas_call(
        functools.partial(_m1_body, nsteps=nsteps),
        grid=(nsteps,),
        in_specs=[_rows(128, 8000), _rows(128, 8000)],
        out_specs=_full((8, 64)),
        out_shape=_fs((8, 64)),
        scratch_shapes=[pltpu.VMEM((8, 64), jnp.float32)],
    )(ga, gb)


def _m2_body(st1_ref, g_ref, be_ref, wt_ref, b_ref, ga_ref, gb_ref,
             y_ref, st_ref, acc_ref, *, nsteps):
    i = pl.program_id(0)

    @pl.when(i == 0)
    def _():
        acc_ref[...] = jnp.zeros_like(acc_ref)

    p = st1_ref[...]
    s1, t1 = _stats_fold(p[0], p[1], float(E), g_ref[0], be_ref[0])
    r1 = jnp.maximum(ga_ref[:, :64] + gb_ref[:, 64:], 0.0)
    wt = wt_ref[...]
    h = jnp.maximum(r1 @ (wt * s1[:, None]) + (t1 @ wt + b_ref[0]), 0.0)
    y_ref[...] = jnp.stack([h[:, :32], h[:, 32:]], axis=0)
    acc_ref[...] += jnp.concatenate(
        [jnp.sum(h, axis=0, keepdims=True),
         jnp.sum(h * h, axis=0, keepdims=True),
         jnp.zeros((6, 64), jnp.float32)], axis=0)

    @pl.when(i == nsteps - 1)
    def _():
        st_ref[...] = acc_ref[...]


def _m2(st1, g1, be1, w2t, b2, ga, gb):
    nsteps = 100
    return pl.pallas_call(
        functools.partial(_m2_body, nsteps=nsteps),
        grid=(nsteps,),
        in_specs=[_full((8, 64)), _full((1, 64)), _full((1, 64)),
                  _full((64, 64)), _full((1, 64)),
                  _rows(128, 8000), _rows(128, 8000)],
        out_specs=[pl.BlockSpec((2, 8000, 32), lambda i: (0, i, 0)),
                   _full((8, 64))],
        out_shape=[_fs((2, E, 32)), _fs((8, 64))],
        scratch_shapes=[pltpu.VMEM((8, 64), jnp.float32)],
    )(st1, g1, be1, w2t, b2, ga, gb)


def _f1_body(x1_ref, x2_ref, acc_ref_in, degp_ref, st_ref_in, g_ref, be_ref,
             wa_ref, wb_ref, wc_ref, b_ref, y_ref, st_ref, acc_ref, *, nsteps):
    i = pl.program_id(0)

    @pl.when(i == 0)
    def _():
        acc_ref[...] = jnp.zeros_like(acc_ref)

    p = st_ref_in[...]
    s2, t2 = _stats_fold(p[0], p[1], float(E), g_ref[0], be_ref[0])
    deg = degp_ref[0, :, 0:1] + degp_ref[1, :, 0:1]
    summed = jnp.concatenate([acc_ref_in[0], acc_ref_in[1]], axis=1)
    x3 = summed * s2 + deg * t2[None, :]
    h = (x1_ref[...] @ wa_ref[...] + x2_ref[...] @ wb_ref[...]
         + x3 @ wc_ref[...] + b_ref[0])
    h = jnp.maximum(h, 0.0)
    y_ref[...] = h
    c = h.shape[1]
    acc_ref[...] += jnp.concatenate(
        [jnp.sum(h, axis=0, keepdims=True),
         jnp.sum(h * h, axis=0, keepdims=True),
         jnp.zeros((6, c), jnp.float32)], axis=0)

    @pl.when(i == nsteps - 1)
    def _():
        st_ref[...] = acc_ref[...]


def _f1(x1, x2, acc3, degp, st2, g2, be2, wa, wb, wc, b):
    return pl.pallas_call(
        functools.partial(_f1_body, nsteps=50),
        grid=(50,),
        in_specs=[_rows(64, 1000), _rows(64, 1000), _acc_spec(32, 1000),
                  _acc_spec(16, 1000),
                  _full((8, 64)), _full((1, 64)), _full((1, 64)),
                  _full((64, 1024)), _full((64, 1024)), _full((64, 1024)),
                  _full((1, 1024))],
        out_specs=[_rows(1024, 1000), _full((8, 1024))],
        out_shape=[_fs((N, 1024)), _fs((8, 1024))],
        scratch_shapes=[pltpu.VMEM((8, 1024), jnp.float32)],
    )(x1, x2, acc3, degp, st2, g2, be2, wa, wb, wc, b)


def _lin_body(st_in_ref, g_ref, be_ref, wt_ref, b_ref, x_ref, y_ref, st_ref,
              acc_ref, *, nsteps):
    i = pl.program_id(0)

    @pl.when(i == 0)
    def _():
        acc_ref[...] = jnp.zeros_like(acc_ref)

    p = st_in_ref[...]
    s, t = _stats_fold(p[0], p[1], float(N), g_ref[0], be_ref[0])
    wt = wt_ref[...]
    h = jnp.maximum(x_ref[...] @ (wt * s[:, None]) + (t @ wt + b_ref[0]), 0.0)
    y_ref[...] = h
    c = h.shape[1]
    acc_ref[...] += jnp.concatenate(
        [jnp.sum(h, axis=0, keepdims=True),
         jnp.sum(h * h, axis=0, keepdims=True),
         jnp.zeros((6, c), jnp.float32)], axis=0)

    @pl.when(i == nsteps - 1)
    def _():
        st_ref[...] = acc_ref[...]


def _lin(st_in, g, be, wt, b, x, cin, cout):
    return pl.pallas_call(
        functools.partial(_lin_body, nsteps=25),
        grid=(25,),
        in_specs=[_full((8, cin)), _full((1, cin)), _full((1, cin)),
                  _full((cin, cout)), _full((1, cout)), _rows(cin, 2000)],
        out_specs=[_rows(cout, 2000), _full((8, cout))],
        out_shape=[_fs((N, cout)), _fs((8, cout))],
        scratch_shapes=[pltpu.VMEM((8, cout), jnp.float32)],
    )(st_in, g, be, wt, b, x)


def _f4_body(st_in_ref, g_ref, be_ref, wt_ref, b_ref, x_ref, o_ref):
    p = st_in_ref[...]
    s, t = _stats_fold(p[0], p[1], float(N), g_ref[0], be_ref[0])
    z = (x_ref[...] * s + t) @ wt_ref[...] + b_ref[0]
    m = jnp.max(z, axis=1, keepdims=True)
    lse = jnp.log(jnp.sum(jnp.exp(z - m), axis=1, keepdims=True)) + m
    o_ref[...] = z - lse


def _f4(st_in, g, be, wt, b, x):
    return pl.pallas_call(
        _f4_body,
        grid=(10,),
        in_specs=[_full((8, 128)), _full((1, 128)), _full((1, 128)),
                  _full((128, 50)), _full((1, 50)), _rows(128, 5000)],
        out_specs=_rows(50, 5000),
        out_shape=_fs((N, 50)),
    )(st_in, g, be, wt, b, x)


def _conv_weights(layers, cin):
    l0, l1 = layers
    w = l0['W']
    wdt = (w[:, :cin] - w[:, cin:]).T
    wst = w[:, cin:].T
    return (wdt, wst, l0['b'].reshape(1, -1), l0['gamma'].reshape(1, -1),
            l0['beta'].reshape(1, -1), l1['W'].T, l1['b'].reshape(1, -1),
            l1['gamma'].reshape(1, -1), l1['beta'].reshape(1, -1))


def _segsum(r2s, dst):
    r2 = jnp.concatenate([r2s[0], r2s[1]], axis=1)
    summed = jax.ops.segment_sum(r2, dst, num_segments=N)
    acc = jnp.zeros((2, SPR, 32), jnp.float32)
    return acc.at[0, :N].set(summed[:, :32]).at[1, :N].set(summed[:, 32:])


def _degrees(dst):
    deg = jax.ops.segment_sum(jnp.ones((E,), jnp.float32), dst, num_segments=N)
    degp = jnp.zeros((2, SPR, 16), jnp.float32)
    return degp.at[0, :N, 0].set(deg)


def kernel(x, edge_index, params):
    src = edge_index[0].astype(jnp.int32)
    dst = edge_index[1].astype(jnp.int32)

    degp = _degrees(dst)

    wdt, wst, b1, g1, be1, w2t, b2, g2, be2 = _conv_weights(params['conv1'], 4)
    tab = _p1(x, wdt, wst, b1)
    ga, gb = _g_call(tab, dst, src)
    st1 = _m1(ga, gb)
    r2s, st2 = _m2(st1, g1, be1, w2t, b2, ga, gb)
    acc1 = _segsum(r2s, dst)

    wdt, wst, b1, g1, be1, w2t, b2, g2_2, be2_2 = _conv_weights(
        params['conv2'], 64)
    x1, tab = _p23(acc1, degp, st2, g2, be2, wdt, wst, b1)
    ga, gb = _g_call(tab, dst, src)
    st1 = _m1(ga, gb)
    r2s, st2 = _m2(st1, g1, be1, w2t, b2, ga, gb)
    acc2 = _segsum(r2s, dst)

    wdt, wst, b1, g1, be1, w2t, b2, g2_3, be2_3 = _conv_weights(
        params['conv3'], 64)
    x2, tab = _p23(acc2, degp, st2, g2_2, be2_2, wdt, wst, b1)
    ga, gb = _g_call(tab, dst, src)
    st1 = _m1(ga, gb)
    r2s, st2 = _m2(st1, g1, be1, w2t, b2, ga, gb)
    acc3 = _segsum(r2s, dst)

    lt = params['lin1'][0]
    wt = lt['W'].T
    h1, stf = _f1(x1, x2, acc3, degp, st2, g2_3, be2_3,
                  wt[0:64], wt[64:128], wt[128:192], lt['b'].reshape(1, -1))
    l = params['mlp1'][0]
    h2, stf2 = _lin(stf, lt['gamma'].reshape(1, -1), lt['beta'].reshape(1, -1),
                    l['W'].T, l['b'].reshape(1, -1), h1, 1024, 256)
    l2 = params['mlp2'][0]
    h3, stf3 = _lin(stf2, l['gamma'].reshape(1, -1), l['beta'].reshape(1, -1),
                    l2['W'].T, l2['b'].reshape(1, -1), h2, 256, 128)
    fin = params['final']
    return _f4(stf3, l2['gamma'].reshape(1, -1), l2['beta'].reshape(1, -1),
               fin['W'].T, fin['b'].reshape(1, -1), h3)
